# Initial kernel scaffold; baseline (speedup 1.0000x reference)
import jax
import jax.numpy as jnp
from jax.experimental import pallas as pl


def kernel(x_scalar, x_vec, pos, edge_index, edge_w, params):
    raise NotImplementedError("write your pallas kernel here")


# speedup vs baseline: 31.4176x; 31.4176x over previous
import functools
import math

import jax
import jax.numpy as jnp
from jax import lax
from jax.experimental import pallas as pl
from jax.experimental.pallas import tpu as pltpu
from jax.experimental.pallas import tpu_sc as plsc

N = 50000
E = 800000
C = 8
H = 16

NP = 50176
EP = 802816
NW = 32
EPW = EP // NW
CH = 512
NCH = EPW // CH
SUB = CH // 128
TS = NP // 16

_f32 = jnp.float32
_i32 = jnp.int32

@functools.lru_cache(maxsize=None)
def _mesh():
    return plsc.VectorSubcoreMesh(core_axis_name="c", subcore_axis_name="s")


_CPARAMS = pltpu.CompilerParams(use_tc_tiling_on_sc=False,
                               needs_layout_passes=False)

_NEG = -1e30


def _wid():
    return lax.axis_index("s") * 2 + lax.axis_index("c")


def _iota16():
    return lax.iota(_i32, 16)


def _cc(c):
    return jnp.full((16,), c, _i32)


def _geom_body(pos4, srcp, dstp, relT, sidx, didx, ps, pd, rb, sem):
    w = _wid()

    def chunk(ch, _):
        base = w * EPW + ch * CH
        cps = []
        for j in range(SUB):
            cps.append(pltpu.async_copy(
                srcp.at[pl.ds(base + j * 128, 128)], sidx.at[j], sem))
            cps.append(pltpu.async_copy(
                dstp.at[pl.ds(base + j * 128, 128)], didx.at[j], sem))
        for cp in cps:
            cp.wait()
        cps = []
        for j in range(SUB):
            cps.append(pltpu.async_copy(
                pos4.at[sidx.at[j]], ps.at[pl.ds(j * 128, 128)], sem))
            cps.append(pltpu.async_copy(
                pos4.at[didx.at[j]], pd.at[pl.ds(j * 128, 128)], sem))
        for cp in cps:
            cp.wait()
        for g in range(CH // 16):
            eid = g * 16 + _iota16()
            for comp in range(3):
                a = plsc.load_gather(pd, [eid, _cc(comp)])
                b = plsc.load_gather(ps, [eid, _cc(comp)])
                rb[comp, pl.ds(g * 16, 16)] = a - b
        for comp in range(3):
            pltpu.sync_copy(rb.at[comp], relT.at[pl.ds(comp * EP + base, CH)])
        return 0

    lax.fori_loop(0, NCH, chunk, 0)


@functools.lru_cache(maxsize=None)
def _geom():
    return pl.kernel(
    _geom_body,
    out_type=[jax.ShapeDtypeStruct((3 * EP,), _f32)],
    mesh=_mesh(),
    compiler_params=_CPARAMS,
    scratch_types=[
        pltpu.VMEM((SUB, 128), _i32),
        pltpu.VMEM((SUB, 128), _i32),
        pltpu.VMEM((CH, 4), _f32),
        pltpu.VMEM((CH, 4), _f32),
        pltpu.VMEM((3, CH), _f32),
        pltpu.SemaphoreType.DMA,
    ],
)


def _passA_body(ktab, qtab, wkT, srcp, dstp, lout, mpart,
                sidx, didx, kb, qb, wkb, lb, mloc, sem):
    w = _wid()

    def init(i, _):
        mloc[pl.ds(i * 16, 16)] = jnp.full((16,), _NEG, _f32)
        return 0

    lax.fori_loop(0, NP // 16, init, 0)

    def chunk(ch, _):
        base = w * EPW + ch * CH
        cps = []
        for j in range(SUB):
            cps.append(pltpu.async_copy(
                srcp.at[pl.ds(base + j * 128, 128)], sidx.at[j], sem))
            cps.append(pltpu.async_copy(
                dstp.at[pl.ds(base + j * 128, 128)], didx.at[j], sem))
        for c in range(C):
            cps.append(pltpu.async_copy(
                wkT.at[pl.ds(c * EP + base, CH)], wkb.at[c], sem))
        for cp in cps:
            cp.wait()
        cps = []
        for j in range(SUB):
            cps.append(pltpu.async_copy(
                ktab.at[sidx.at[j]], kb.at[pl.ds(j * 128, 128)], sem))
            cps.append(pltpu.async_copy(
                qtab.at[didx.at[j]], qb.at[pl.ds(j * 128, 128)], sem))
        for cp in cps:
            cp.wait()
        scale = 1.0 / math.sqrt(float(C))
        for g in range(CH // 16):
            eid = g * 16 + _iota16()
            acc = jnp.zeros((16,), _f32)
            for c in range(C):
                acc += (plsc.load_gather(kb, [eid, _cc(c)])
                        * plsc.load_gather(qb, [eid, _cc(c)])
                        * wkb[c, pl.ds(g * 16, 16)])
            l = acc * scale
            lb[pl.ds(g * 16, 16)] = l
            d = didx[g // 8, pl.ds((g % 8) * 16, 16)]

            def mcond(act):
                return jnp.any(act)

            def mbody(act):
                plsc.store_scatter(mloc, [d], l, mask=act)
                cur = plsc.load_gather(mloc, [d])
                return cur < l

            lax.while_loop(mcond, mbody, jnp.full((16,), True))
        pltpu.sync_copy(lb, lout.at[pl.ds(base, CH)])
        return 0

    lax.fori_loop(0, NCH, chunk, 0)

    pltpu.sync_copy(mloc, mpart.at[pl.ds(w * NP, NP)])


@functools.lru_cache(maxsize=None)
def _passA():
    return pl.kernel(
    _passA_body,
    out_type=[jax.ShapeDtypeStruct((EP,), _f32),
              jax.ShapeDtypeStruct((NW * NP,), _f32)],
    mesh=_mesh(),
    compiler_params=_CPARAMS,
    scratch_types=[
        pltpu.VMEM((SUB, 128), _i32),
        pltpu.VMEM((SUB, 128), _i32),
        pltpu.VMEM((CH, C), _f32),
        pltpu.VMEM((CH, C), _f32),
        pltpu.VMEM((C, CH), _f32),
        pltpu.VMEM((CH,), _f32),
        pltpu.VMEM((NP,), _f32),
        pltpu.SemaphoreType.DMA,
    ],
)


def _passB_body(lin, mtab, dstp, zin, eout, zpart,
                didx, lb, mb, eb, zsh, sem):
    cid = lax.axis_index("c")
    sid = lax.axis_index("s")

    pltpu.sync_copy(zin, zsh.at[pl.ds(sid * TS, TS)])
    plsc.subcore_barrier()

    w = _wid()

    def chunk(ch, _):
        base = w * EPW + ch * CH
        cps = [pltpu.async_copy(lin.at[pl.ds(base, CH)], lb, sem)]
        for j in range(SUB):
            cps.append(pltpu.async_copy(
                dstp.at[pl.ds(base + j * 128, 128)], didx.at[j], sem))
        for cp in cps:
            cp.wait()
        cps = []
        for j in range(SUB):
            cps.append(pltpu.async_copy(
                mtab.at[didx.at[j]], mb.at[pl.ds(j * 128, 128)], sem))
        for cp in cps:
            cp.wait()
        for g in range(CH // 16):
            s = pl.ds(g * 16, 16)
            eb[s] = jnp.exp(lb[s] - mb[s])
        pltpu.sync_copy(eb, eout.at[pl.ds(base, CH)])
        for j in range(SUB):
            pltpu.sync_copy(eb.at[pl.ds(j * 128, 128)],
                            zsh.at[didx.at[j]], add=True)
        return 0

    lax.fori_loop(0, NCH, chunk, 0)
    plsc.subcore_barrier()
    pltpu.sync_copy(zsh.at[pl.ds(sid * TS, TS)],
                    zpart.at[pl.ds(cid * NP + sid * TS, TS)])


@functools.lru_cache(maxsize=None)
def _passB():
    return pl.kernel(
    _passB_body,
    out_type=[jax.ShapeDtypeStruct((EP,), _f32),
              jax.ShapeDtypeStruct((2 * NP,), _f32)],
    mesh=_mesh(),
    compiler_params=_CPARAMS,
    scratch_types=[
        pltpu.VMEM((SUB, 128), _i32),
        pltpu.VMEM((CH,), _f32),
        pltpu.VMEM((CH,), _f32),
        pltpu.VMEM((CH,), _f32),
        pltpu.VMEM_SHARED((NP,), _f32),
        pltpu.SemaphoreType.DMA,
    ],
)


def _passC_body(ein, ztab, vtab, wrT, rhT, srcp, dstp, zin, accpart,
                sidx, didx, eb, zb, vb, wrb, rb, ub, accsh, sem):
    VW = 20
    W = 16
    R = 12
    cid = lax.axis_index("c")
    sid = lax.axis_index("s")

    pltpu.sync_copy(zin, accsh.at[pl.ds(sid * TS, TS)])
    plsc.subcore_barrier()

    w = _wid()

    def chunk(ch, _):
        base = w * EPW + ch * CH
        cps = [pltpu.async_copy(ein.at[pl.ds(base, CH)], eb, sem)]
        for j in range(SUB):
            cps.append(pltpu.async_copy(
                srcp.at[pl.ds(base + j * 128, 128)], sidx.at[j], sem))
            cps.append(pltpu.async_copy(
                dstp.at[pl.ds(base + j * 128, 128)], didx.at[j], sem))
        for r in range(R):
            cps.append(pltpu.async_copy(
                wrT.at[pl.ds(r * EP + base, CH)], wrb.at[r], sem))
        for comp in range(3):
            cps.append(pltpu.async_copy(
                rhT.at[pl.ds(comp * EP + base, CH)], rb.at[comp], sem))
        for cp in cps:
            cp.wait()
        cps = []
        for j in range(SUB):
            cps.append(pltpu.async_copy(
                vtab.at[sidx.at[j]], vb.at[pl.ds(j * 128, 128)], sem))
            cps.append(pltpu.async_copy(
                ztab.at[didx.at[j]], zb.at[pl.ds(j * 128, 128)], sem))
        for cp in cps:
            cp.wait()
        for g in range(CH // 16):
            s = pl.ds(g * 16, 16)
            eid = g * 16 + _iota16()
            alpha = eb[s] / (zb[s] + 1e-9)
            for c in range(4):
                wv1c = wrb[c, s]
                wxc = wrb[4 + c, s]
                for i in range(3):
                    v = (plsc.load_gather(vb, [eid, _cc(4 * c + i)])
                         * wv1c + wxc * rb[i, s])
                    plsc.store_scatter(ub, [eid, _cc(3 * c + i)],
                                       alpha * v)
            for k in range(4):
                v0 = plsc.load_gather(vb, [eid, _cc(16 + k)])                     * wrb[8 + k, s]
                plsc.store_scatter(ub, [eid, _cc(12 + k)], alpha * v0)
        for j in range(SUB):
            pltpu.sync_copy(ub.at[pl.ds(j * 128, 128)],
                            accsh.at[didx.at[j]], add=True)
        return 0

    lax.fori_loop(0, NCH, chunk, 0)
    plsc.subcore_barrier()
    pltpu.sync_copy(accsh.at[pl.ds(sid * TS, TS)],
                    accpart.at[pl.ds(cid * NP + sid * TS, TS)])


@functools.lru_cache(maxsize=None)
def _passC():
    return pl.kernel(
        _passC_body,
        out_type=[jax.ShapeDtypeStruct((2 * NP, 16), _f32)],
        mesh=_mesh(),
        compiler_params=_CPARAMS,
        scratch_types=[
            pltpu.VMEM((SUB, 128), _i32),
            pltpu.VMEM((SUB, 128), _i32),
            pltpu.VMEM((CH,), _f32),
            pltpu.VMEM((CH,), _f32),
            pltpu.VMEM((CH, 20), _f32),
            pltpu.VMEM((12, CH), _f32),
            pltpu.VMEM((3, CH), _f32),
            pltpu.VMEM((CH, 16), _f32),
            pltpu.VMEM_SHARED((NP, 16), _f32),
            pltpu.SemaphoreType.DMA,
        ],
    )


EPR = EP // 128


@functools.lru_cache(maxsize=None)
def _geomtc():
    def body(rel_ref, dist_ref, rh_ref):
        x = rel_ref[0]
        y = rel_ref[1]
        z = rel_ref[2]
        d = jnp.sqrt(x * x + y * y + z * z)
        inv = 1.0 / (d + 1e-8)
        dist_ref[...] = d
        rh_ref[0] = x * inv
        rh_ref[1] = y * inv
        rh_ref[2] = z * inv

    blk = 64
    return pl.pallas_call(
        body,
        grid=(EPR // blk,),
        in_specs=[pl.BlockSpec((3, blk, 128), lambda i: (0, i, 0))],
        out_specs=[pl.BlockSpec((blk, 128), lambda i: (i, 0)),
                   pl.BlockSpec((3, blk, 128), lambda i: (0, i, 0))],
        out_shape=[jax.ShapeDtypeStruct((EPR, 128), _f32),
                   jax.ShapeDtypeStruct((3, EPR, 128), _f32)],
    )


@functools.lru_cache(maxsize=None)
def _radialtc(O):
    def body(d_ref, e_ref, w1_ref, b1_ref, w2_ref, b2_ref, out_ref):
        d = d_ref[...]
        e = e_ref[...]
        rs = []
        for h in range(H):
            rs.append(jax.nn.relu(d * w1_ref[0, h] + e * w1_ref[1, h]
                                  + b1_ref[h]))
        for o in range(O):
            acc = jnp.full(d.shape, b2_ref[o], _f32)
            for h in range(H):
                acc = acc + rs[h] * w2_ref[h, o]
            out_ref[o] = acc

    blk = 32
    smem = pl.BlockSpec(memory_space=pltpu.SMEM)
    return pl.pallas_call(
        body,
        grid=(EPR // blk,),
        in_specs=[pl.BlockSpec((blk, 128), lambda i: (i, 0)),
                  pl.BlockSpec((blk, 128), lambda i: (i, 0)),
                  smem, smem, smem, smem],
        out_specs=[pl.BlockSpec((O, blk, 128), lambda i: (0, i, 0))],
        out_shape=[jax.ShapeDtypeStruct((O, EPR, 128), _f32)],
    )


@functools.lru_cache(maxsize=None)
def _reduce_tc(rows, m, op):
    def body(x_ref, o_ref):
        x = x_ref[...]
        if op == 'max':
            o_ref[...] = jnp.max(x, axis=0)
        else:
            o_ref[...] = jnp.sum(x, axis=0)

    return pl.pallas_call(
        body,
        grid=(m // 1024,),
        in_specs=[pl.BlockSpec((rows, 1024), lambda i: (0, i))],
        out_specs=[pl.BlockSpec((1024,), lambda i: (i,))],
        out_shape=[jax.ShapeDtypeStruct((m,), _f32)],
    )


def _reduce(x, op):
    rows, m = x.shape
    out, = _reduce_tc(rows, m, op)(x)
    return out


@functools.lru_cache(maxsize=None)
def _tablestc():
    def body(h0_ref, h1_ref, wq_ref, wk_ref, wv0_ref, m32_ref,
             q_ref, k_ref, v0_ref, v1_ref):
        h0 = h0_ref[...]
        h1 = h1_ref[...]
        q_ref[...] = jnp.dot(h0, wq_ref[...])
        k_ref[...] = jnp.dot(h0, wk_ref[...])
        v0_ref[...] = jnp.dot(h0, wv0_ref[...])
        v1_ref[...] = jnp.dot(h1, m32_ref[...])

    blk = 512
    return pl.pallas_call(
        body,
        grid=(NP // blk,),
        in_specs=[pl.BlockSpec((blk, 16), lambda i: (i, 0)),
                  pl.BlockSpec((blk, 32), lambda i: (i, 0)),
                  pl.BlockSpec((16, 8), lambda i: (0, 0)),
                  pl.BlockSpec((16, 8), lambda i: (0, 0)),
                  pl.BlockSpec((16, 8), lambda i: (0, 0)),
                  pl.BlockSpec((32, 32), lambda i: (0, 0))],
        out_specs=[pl.BlockSpec((blk, 8), lambda i: (i, 0)),
                   pl.BlockSpec((blk, 8), lambda i: (i, 0)),
                   pl.BlockSpec((blk, 8), lambda i: (i, 0)),
                   pl.BlockSpec((blk, 32), lambda i: (i, 0))],
        out_shape=[jax.ShapeDtypeStruct((NP, 8), _f32),
                   jax.ShapeDtypeStruct((NP, 8), _f32),
                   jax.ShapeDtypeStruct((NP, 8), _f32),
                   jax.ShapeDtypeStruct((NP, 32), _f32)],
    )


@functools.lru_cache(maxsize=None)
def _updatetc():
    def body(x0_ref, x1_ref, w16_ref, m64_ref, s_ref, st_ref,
             bc_ref, bn1_ref, h0_ref, h1_ref):
        h0_ref[...] = jax.nn.relu(jnp.dot(x0_ref[...], w16_ref[...])
                                  + bc_ref[...])
        y = jnp.dot(x1_ref[...], m64_ref[...])[:, :32]
        n2 = jnp.dot(y * y, s_ref[...])
        nrm = jnp.sqrt(n2)
        sc = jax.nn.relu(nrm + bn1_ref[...]) / (nrm + 1e-8)
        h1_ref[...] = y * jnp.dot(sc, st_ref[...])

    blk = 512
    return pl.pallas_call(
        body,
        grid=(NP // blk,),
        in_specs=[pl.BlockSpec((blk, 16), lambda i: (i, 0)),
                  pl.BlockSpec((blk, 64), lambda i: (i, 0)),
                  pl.BlockSpec((16, 8), lambda i: (0, 0)),
                  pl.BlockSpec((64, 64), lambda i: (0, 0)),
                  pl.BlockSpec((32, 8), lambda i: (0, 0)),
                  pl.BlockSpec((8, 32), lambda i: (0, 0)),
                  pl.BlockSpec((1, 8), lambda i: (0, 0)),
                  pl.BlockSpec((1, 8), lambda i: (0, 0))],
        out_specs=[pl.BlockSpec((blk, 8), lambda i: (i, 0)),
                   pl.BlockSpec((blk, 32), lambda i: (i, 0))],
        out_shape=[jax.ShapeDtypeStruct((NP, 8), _f32),
                   jax.ShapeDtypeStruct((NP, 32), _f32)],
    )


@functools.lru_cache(maxsize=None)
def _finaltc():
    def body(x_ref, m40_ref, o_ref):
        o_ref[...] = jnp.dot(x_ref[...], m40_ref[...])

    blk = 512
    return pl.pallas_call(
        body,
        grid=(NP // blk,),
        in_specs=[pl.BlockSpec((blk, 40), lambda i: (i, 0)),
                  pl.BlockSpec((40, 8), lambda i: (0, 0))],
        out_specs=[pl.BlockSpec((blk, 8), lambda i: (i, 0))],
        out_shape=[jax.ShapeDtypeStruct((NP, 8), _f32)],
    )


def _mk_m32(wv1):
    cin = wv1.shape[0]
    m = jnp.zeros((32, 32), _f32)
    for k in range(cin):
        for c in range(8 if wv1.shape[1] == 8 else wv1.shape[1]):
            for i in range(3):
                m = m.at[4 * k + i, 4 * c + i].set(wv1[k, c])
    return m


def _mk_m64(wsi1, cin):
    mm = jnp.zeros((64, 64), _f32)
    for k in range(cin):
        for c in range(8):
            for i in range(3):
                mm = mm.at[4 * k + i, 4 * c + i].set(wsi1[k, c])
    for k2 in range(8):
        for c in range(8):
            for i in range(3):
                mm = mm.at[4 * cin + 4 * k2 + i, 4 * c + i].set(
                    wsi1[cin + k2, c])
    return mm


def _mk_m40(wsi1f):
    mm = jnp.zeros((40, 8), _f32)
    for k in range(10):
        for c in range(2):
            for i in range(3):
                mm = mm.at[4 * k + i, 4 * c + i].set(wsi1f[k, c])
    return mm


_S32 = None

import numpy as _np

_S_NP = _np.zeros((32, 8), _np.float32)
_ST_NP = _np.zeros((8, 32), _np.float32)
for _k in range(8):
    for _i in range(4):
        _S_NP[4 * _k + _i, _k] = 1.0
        _ST_NP[_k, 4 * _k + _i] = 1.0


def _pad2(x, cols):
    return jnp.pad(x, ((0, 0), (0, cols - x.shape[1])))


def _zeros1():
    return jnp.zeros((TS,), _f32)


def _zeros2(w):
    return jnp.zeros((TS, w), _f32)


def kernel(x_scalar, x_vec, pos, edge_index, edge_w, params):
    src = edge_index[0]
    dst = edge_index[1]
    pe = EP - E
    trash = (50000 + (jnp.arange(pe, dtype=_i32) % 128)).astype(_i32)
    srcp = jnp.concatenate([src, trash])
    dstp = jnp.concatenate([dst, trash])
    ewp = jnp.concatenate([edge_w[:, 0], jnp.zeros((pe,), _f32)])
    pos4 = jnp.pad(pos, ((0, NP - N), (0, 1)))

    relT, = _geom()(pos4, srcp, dstp)
    dist2, rh3 = _geomtc()(relT.reshape(3, EPR, 128))
    rhTf = rh3.reshape(-1)
    ew2 = ewp.reshape(EPR, 128)

    S = jnp.asarray(_S_NP)
    ST = jnp.asarray(_ST_NP)

    h0 = jnp.pad(x_scalar, ((0, NP - N), (0, 0)))
    h1pp = jnp.pad(x_vec, ((0, NP - N), (0, 1)))
    cin = 1

    for p in params['layers']:
        rw3, = _radialtc(4 * C)(dist2, ew2, p['Wr1'], p['br1'],
                                p['Wr2'], p['br2'])
        rwT = rw3.reshape(4 * C, EP)
        wkT = rwT[:C].reshape(-1)

        q, k, v0, v1p = _tablestc()(
            _pad2(h0, 16), _pad2(h1pp, 32),
            _pad2(jnp.pad(p['Wq'], ((0, 16 - cin), (0, 0))), 8),
            jnp.pad(p['Wk'], ((0, 16 - cin), (0, 0))),
            jnp.pad(p['Wv0'], ((0, 16 - cin), (0, 0))),
            _mk_m32(p['Wv1']))

        lvec, mpart = _passA()(k, q, wkT, srcp, dstp)
        mtab = _reduce(mpart.reshape(NW, NP), 'max')
        evec, zpart = _passB()(lvec, mtab, dstp, _zeros1())
        ztab = _reduce(zpart.reshape(2, NP), 'sum')
        m0s, m1s = [], []
        for t in range(2):
            vtab_t = jnp.concatenate(
                [v1p[:, 16 * t:16 * t + 16], v0[:, 4 * t:4 * t + 4]], axis=1)
            wr_t = jnp.concatenate(
                [rwT[2 * C + 4 * t:2 * C + 4 * t + 4],
                 rwT[3 * C + 4 * t:3 * C + 4 * t + 4],
                 rwT[C + 4 * t:C + 4 * t + 4]], axis=0).reshape(-1)
            accpart, = _passC()(evec, ztab, vtab_t, wr_t, rhTf,
                                srcp, dstp, _zeros2(16))
            acc_t = _reduce(accpart.reshape(2, NP * 16), 'sum')                 .reshape(NP, 16)
            m1s.append(acc_t[:, :12].reshape(NP, 4, 3))
            m0s.append(acc_t[:, 12:16])
        m0 = jnp.concatenate(m0s, axis=1)
        m1 = jnp.concatenate(m1s, axis=1)
        m1pp = jnp.pad(m1, ((0, 0), (0, 0), (0, 1))).reshape(NP, 32)

        x0 = _pad2(jnp.concatenate([h0, m0], axis=1), 16)
        x1 = _pad2(jnp.concatenate([h1pp, m1pp], axis=1), 64)
        w16 = jnp.pad(p['Wsi0'], ((0, 16 - (cin + C)), (0, 0)))
        m64 = _mk_m64(p['Wsi1'], cin)
        bc = (p['bsi0'] + p['bn0'])[None, :]
        h0, h1pp = _updatetc()(x0, x1, w16, m64, S, ST, bc,
                               p['bn1'][None, :])
        cin = C

    p = params['final']
    rw3, = _radialtc(C + 4)(dist2, ew2, p['Wr1'], p['br1'],
                            p['Wr2'], p['br2'])
    rwT12 = rw3.reshape(C + 4, EP)
    wkT = rwT12[:C].reshape(-1)

    q, k, _v0u, v1p = _tablestc()(
        _pad2(h0, 16), h1pp,
        jnp.pad(p['Wq'], ((0, 8), (0, 0))),
        jnp.pad(p['Wk'], ((0, 8), (0, 0))),
        jnp.pad(p['Wk'], ((0, 8), (0, 0))),
        _mk_m32(p['Wv1']))
    vtab_f = _pad2(v1p[:, :8], 20)

    wr_f = jnp.zeros((12, EP), _f32)
    wr_f = wr_f.at[0:2].set(rwT12[C:C + 2])
    wr_f = wr_f.at[4:6].set(rwT12[C + 2:C + 4])
    wr_f = wr_f.reshape(-1)

    lvec, mpart = _passA()(k, q, wkT, srcp, dstp)
    mtab = _reduce(mpart.reshape(NW, NP), 'max')
    evec, zpart = _passB()(lvec, mtab, dstp, _zeros1())
    ztab = _reduce(zpart.reshape(2, NP), 'sum')
    accpart, = _passC()(evec, ztab, vtab_f, wr_f, rhTf, srcp, dstp,
                        _zeros2(16))
    accf = _reduce(accpart.reshape(2, NP * 16), 'sum').reshape(NP, 16)

    m1f = accf[:, :6].reshape(NP, 2, 3)
    m1fp8 = jnp.pad(m1f, ((0, 0), (0, 0), (0, 1))).reshape(NP, 8)
    xf = jnp.concatenate([h1pp, m1fp8], axis=1)
    yf, = _finaltc()(xf, _mk_m40(p['Wsi1']))
    return yf[:N].reshape(N, 2, 4)[:, :, :3]
